# baseline (speedup 1.0000x reference)
import functools

import jax
import jax.numpy as jnp
from jax import lax
from jax.experimental import pallas as pl
from jax.experimental.pallas import tpu as pltpu
from jax.experimental.pallas import tpu_sc as plsc

N_NODES = 10000
N_PAD = 10240
N_EDGES = 320000
D_HID = 128
N_CLASSES = 21
D_OUT_PAD = 32

NC = 2
NS = 16
NW = NC * NS
E_TILE = N_EDGES // NW
K = 80
NCHUNK = E_TILE // K
KD = 400
NCHUNK_D = E_TILE // KD
ROWS_TILE = N_PAD // NS
ZROWS = 32

@functools.cache
def _mesh():
  return plsc.VectorSubcoreMesh(core_axis_name="c", subcore_axis_name="s",
                                num_cores=NC, num_subcores=NS)


def _zero_vmem_2d(ref, nrows, ncols):
  z16 = jnp.zeros((16,), jnp.float32)

  def body(j, carry):
    for q in range(ncols // 16):
      ref[j, pl.ds(q * 16, 16)] = z16
    return carry

  lax.fori_loop(0, nrows, body, 0)


def _deg_body(src_hbm, dst_hbm, dego_hbm, degi_hbm,
              acc_o, acc_i, src_v, dst_v, ones_v, zbuf):
  c = lax.axis_index("c")
  s = lax.axis_index("s")
  wid = s * NC + c

  _zero_vmem_2d(zbuf, ZROWS, 16)
  o16 = jnp.ones((16,), jnp.float32)

  def fill_ones(j, carry):
    ones_v[j, pl.ds(0, 16)] = o16
    return carry

  lax.fori_loop(0, KD, fill_ones, 0)

  for r in range(ROWS_TILE // ZROWS):
    base = s * ROWS_TILE + r * ZROWS
    pltpu.sync_copy(zbuf, acc_o.at[pl.ds(base, ZROWS)])
    pltpu.sync_copy(zbuf, acc_i.at[pl.ds(base, ZROWS)])
  plsc.subcore_barrier()

  pltpu.sync_copy(src_hbm.at[wid], src_v)
  pltpu.sync_copy(dst_hbm.at[wid], dst_v)

  def chunk(j, carry):
    pltpu.sync_copy(ones_v, acc_o.at[src_v.at[j]], add=True)
    pltpu.sync_copy(ones_v, acc_i.at[dst_v.at[j]], add=True)
    return carry

  lax.fori_loop(0, NCHUNK_D, chunk, 0)
  plsc.subcore_barrier()

  base = s * ROWS_TILE
  pltpu.sync_copy(acc_o.at[pl.ds(base, ROWS_TILE)],
                  dego_hbm.at[c].at[pl.ds(base, ROWS_TILE)])
  pltpu.sync_copy(acc_i.at[pl.ds(base, ROWS_TILE)],
                  degi_hbm.at[c].at[pl.ds(base, ROWS_TILE)])


@functools.cache
def _degrees_sc():
  return pl.kernel(
      _deg_body,
      out_type=(
          jax.ShapeDtypeStruct((NC, N_PAD, 16), jnp.float32),
          jax.ShapeDtypeStruct((NC, N_PAD, 16), jnp.float32),
      ),
      mesh=_mesh(),
      scratch_types=[
          pltpu.VMEM_SHARED((N_PAD, 16), jnp.float32),
          pltpu.VMEM_SHARED((N_PAD, 16), jnp.float32),
          pltpu.VMEM((NCHUNK_D, KD), jnp.int32),
          pltpu.VMEM((NCHUNK_D, KD), jnp.int32),
          pltpu.VMEM((KD, 16), jnp.float32),
          pltpu.VMEM((ZROWS, 16), jnp.float32),
      ],
      compiler_params=pltpu.CompilerParams(use_tc_tiling_on_sc=False),
      name="gcn_degrees",
  )


def _agg_body(width, k, nchunk, nbuf, m_hbm, src_hbm, dst_hbm, z_hbm,
              out_hbm, acc, src_v, dst_v, rows, *sems):
  c = lax.axis_index("c")
  s = lax.axis_index("s")
  wid = s * NC + c

  for r in range(ROWS_TILE // ZROWS):
    base = s * ROWS_TILE + r * ZROWS
    pltpu.sync_copy(z_hbm, acc.at[pl.ds(base, ZROWS)])
  plsc.subcore_barrier()

  pltpu.sync_copy(src_hbm.at[wid], src_v)
  pltpu.sync_copy(dst_hbm.at[wid], dst_v)

  gsems = sems[0:2]
  ssems = sems[2:4]
  kh = k // 2
  dummy = m_hbm.at[pl.ds(0, kh)]

  def issue_g(h, b):
    pltpu.async_copy(m_hbm.at[src_v.at[h, 0]], rows.at[b, 0], gsems[b])
    pltpu.async_copy(m_hbm.at[src_v.at[h, 1]], rows.at[b, 1], gsems[b])

  def issue_s(h, b):
    pltpu.async_copy(rows.at[b, 0], acc.at[dst_v.at[h, 0]], ssems[b],
                     add=True)
    pltpu.async_copy(rows.at[b, 1], acc.at[dst_v.at[h, 1]], ssems[b],
                     add=True)

  def wait_g(b):
    pltpu.make_async_copy(dummy, rows.at[b, 0], gsems[b]).wait()
    pltpu.make_async_copy(dummy, rows.at[b, 1], gsems[b]).wait()

  def wait_s(b):
    pltpu.make_async_copy(dummy, rows.at[b, 0], ssems[b]).wait()
    pltpu.make_async_copy(dummy, rows.at[b, 1], ssems[b]).wait()

  issue_g(0, 0)
  issue_g(1, 1)
  wait_g(0)
  issue_s(0, 0)

  def slot_pair(i, carry):
    for h, b in ((2 * i + 1, 1), (2 * i + 2, 0)):
      nb = 1 - b

      @pl.when(h + 1 < nchunk)
      def _():
        wait_s(nb)
        issue_g(h + 1, nb)

      wait_g(b)
      issue_s(h, b)
    return carry

  lax.fori_loop(0, (nchunk - 1) // 2, slot_pair, 0)
  wait_s(1)
  wait_s(0)
  plsc.subcore_barrier()

  base = s * ROWS_TILE
  pltpu.sync_copy(acc.at[pl.ds(base, ROWS_TILE)],
                  out_hbm.at[c].at[pl.ds(base, ROWS_TILE)])


@functools.cache
def _make_agg(width, k, nchunk, nbuf, dtype):
  return pl.kernel(
      functools.partial(_agg_body, width, k, nchunk, nbuf),
      out_type=jax.ShapeDtypeStruct((NC, N_PAD, width), dtype),
      mesh=_mesh(),
      scratch_types=[
          pltpu.VMEM_SHARED((N_PAD, width), dtype),
          pltpu.VMEM((nchunk, 2, k // 2), jnp.int32),
          pltpu.VMEM((nchunk, 2, k // 2), jnp.int32),
          pltpu.VMEM((nbuf, 2, k // 2, width), dtype),
      ] + [pltpu.SemaphoreType.DMA] * (2 * nbuf),
      compiler_params=pltpu.CompilerParams(use_tc_tiling_on_sc=False),
      name=f"gcn_agg_{width}",
  )


_RB = 2000
_GRID = N_NODES // _RB


def _prep_tc(dego_ref, degi_ref, x_ref, s0_ref, nsrc_ref, ndst_ref):
  deg_o = dego_ref[0, :, 0:1] + dego_ref[1, :, 0:1]
  deg_i = degi_ref[0, :, 0:1] + degi_ref[1, :, 0:1]
  nsrc = 1.0 / jnp.sqrt(jnp.maximum(deg_o, 1.0))
  ndst = 1.0 / jnp.sqrt(jnp.maximum(deg_i, 1.0))
  nsrc_ref[...] = nsrc
  ndst_ref[...] = ndst
  s0_ref[...] = x_ref[...] * nsrc


def _mid_tc(p_ref, nsrc_ref, ndst_ref, w_ref, b_ref, s_ref):
  p = p_ref[0].astype(jnp.float32) + p_ref[1].astype(jnp.float32)
  h = p * ndst_ref[...]
  y = jnp.dot(h, w_ref[...], preferred_element_type=jnp.float32) + b_ref[...]
  s_ref[...] = jnp.maximum(y, 0.0) * nsrc_ref[...]


def _last_mid_tc(p_ref, nsrc_ref, ndst_ref, w_ref, b_ref, w5_ref, t_ref):
  p = p_ref[0].astype(jnp.float32) + p_ref[1].astype(jnp.float32)
  h = p * ndst_ref[...]
  y = jnp.dot(h, w_ref[...], preferred_element_type=jnp.float32) + b_ref[...]
  s = jnp.maximum(y, 0.0) * nsrc_ref[...]
  t_ref[...] = jnp.dot(s, w5_ref[...], preferred_element_type=jnp.float32)


def _final_tc(p_ref, ndst_ref, b5_ref, out_ref):
  o = (p_ref[0] + p_ref[1]) * ndst_ref[...]
  out_ref[...] = o[:, :N_CLASSES] + b5_ref[...]


def _row_spec(width):
  return pl.BlockSpec((_RB, width), lambda i: (i, 0))


def _part_spec(width):
  return pl.BlockSpec((NC, _RB, width), lambda i: (0, i, 0))


def _full_spec(shape):
  return pl.BlockSpec(shape, lambda i: tuple(0 for _ in shape))


def kernel(features, edge_index, W1, b1, W2, b2, W3, b3, W4, b4, W5, b5):
  src32 = edge_index[0].astype(jnp.int32)
  dst32 = edge_index[1].astype(jnp.int32)
  src = src32.reshape(NW, NCHUNK, 2, K // 2)
  dst = dst32.reshape(NW, NCHUNK, 2, K // 2)
  src_d = src32.reshape(NW, NCHUNK_D, KD)
  dst_d = dst32.reshape(NW, NCHUNK_D, KD)
  src_d2 = src32.reshape(NW, NCHUNK_D, 2, KD // 2)
  dst_d2 = dst32.reshape(NW, NCHUNK_D, 2, KD // 2)

  dego, degi = _degrees_sc()(src_d, dst_d)

  s0, nsrc, ndst = pl.pallas_call(
      _prep_tc,
      grid=(_GRID,),
      in_specs=[_part_spec(16), _part_spec(16), _row_spec(D_HID)],
      out_specs=[_row_spec(D_HID), _row_spec(1), _row_spec(1)],
      out_shape=[
          jax.ShapeDtypeStruct((N_NODES, D_HID), jnp.float32),
          jax.ShapeDtypeStruct((N_NODES, 1), jnp.float32),
          jax.ShapeDtypeStruct((N_NODES, 1), jnp.float32),
      ],
  )(dego, degi, features)

  mid = pl.pallas_call(
      _mid_tc,
      grid=(_GRID,),
      in_specs=[
          _part_spec(D_HID), _row_spec(1), _row_spec(1),
          _full_spec((D_HID, D_HID)), _full_spec((1, D_HID)),
      ],
      out_specs=_row_spec(D_HID),
      out_shape=jax.ShapeDtypeStruct((N_NODES, D_HID), jnp.float32),
  )

  last_mid = pl.pallas_call(
      _last_mid_tc,
      grid=(_GRID,),
      in_specs=[
          _part_spec(D_HID), _row_spec(1), _row_spec(1),
          _full_spec((D_HID, D_HID)), _full_spec((1, D_HID)),
          _full_spec((D_HID, D_OUT_PAD)),
      ],
      out_specs=_row_spec(D_OUT_PAD),
      out_shape=jax.ShapeDtypeStruct((N_NODES, D_OUT_PAD), jnp.float32),
  )

  W5p = jnp.zeros((D_HID, D_OUT_PAD), jnp.float32).at[:, :N_CLASSES].set(W5)
  z128 = jnp.zeros((ZROWS, D_HID), jnp.float32)
  z32 = jnp.zeros((ZROWS, D_OUT_PAD), jnp.float32)

  agg128 = _make_agg(D_HID, K, NCHUNK, 2, jnp.float32)
  agg32 = _make_agg(D_OUT_PAD, KD, NCHUNK_D, 2, jnp.float32)
  p1 = agg128(s0, src, dst, z128)
  s1 = mid(p1, nsrc, ndst, W1, b1.reshape(1, -1))
  p2 = agg128(s1, src, dst, z128)
  s2 = mid(p2, nsrc, ndst, W2, b2.reshape(1, -1))
  p3 = agg128(s2, src, dst, z128)
  s3 = mid(p3, nsrc, ndst, W3, b3.reshape(1, -1))
  p4 = agg128(s3, src, dst, z128)
  t = last_mid(p4, nsrc, ndst, W4, b4.reshape(1, -1), W5p)
  p5 = agg32(t, src_d2, dst_d2, z32)

  out = pl.pallas_call(
      _final_tc,
      grid=(_GRID,),
      in_specs=[
          _part_spec(D_OUT_PAD), _row_spec(1),
          _full_spec((1, N_CLASSES)),
      ],
      out_specs=_row_spec(N_CLASSES),
      out_shape=jax.ShapeDtypeStruct((N_NODES, N_CLASSES), jnp.float32),
  )(p5, ndst, b5.reshape(1, -1))
  return out


# speedup vs baseline: 1.1648x; 1.1648x over previous
import functools

import jax
import jax.numpy as jnp
from jax import lax
from jax.experimental import pallas as pl
from jax.experimental.pallas import tpu as pltpu
from jax.experimental.pallas import tpu_sc as plsc

N_NODES = 10000
N_PAD = 10240
N_EDGES = 320000
D_HID = 128
N_CLASSES = 21
D_OUT_PAD = 32

NC = 2
NS = 16
NW = NC * NS
E_TILE = N_EDGES // NW
K = 80
NCHUNK = E_TILE // K
KD = 400
NCHUNK_D = E_TILE // KD
ROWS_TILE = N_PAD // NS
ZROWS = 32

@functools.cache
def _mesh():
  return plsc.VectorSubcoreMesh(core_axis_name="c", subcore_axis_name="s",
                                num_cores=NC, num_subcores=NS)


def _zero_vmem_2d(ref, nrows, ncols):
  z16 = jnp.zeros((16,), jnp.float32)

  def body(j, carry):
    for q in range(ncols // 16):
      ref[j, pl.ds(q * 16, 16)] = z16
    return carry

  lax.fori_loop(0, nrows, body, 0)


def _deg_body(src_hbm, dst_hbm, dego_hbm, degi_hbm,
              acc_o, acc_i, src_v, dst_v, ones_v, zbuf):
  c = lax.axis_index("c")
  s = lax.axis_index("s")
  wid = s * NC + c

  _zero_vmem_2d(zbuf, ZROWS, 16)
  o16 = jnp.ones((16,), jnp.float32)

  def fill_ones(j, carry):
    ones_v[j, pl.ds(0, 16)] = o16
    return carry

  lax.fori_loop(0, KD, fill_ones, 0)

  for r in range(ROWS_TILE // ZROWS):
    base = s * ROWS_TILE + r * ZROWS
    pltpu.sync_copy(zbuf, acc_o.at[pl.ds(base, ZROWS)])
    pltpu.sync_copy(zbuf, acc_i.at[pl.ds(base, ZROWS)])
  plsc.subcore_barrier()

  pltpu.sync_copy(src_hbm.at[wid], src_v)
  pltpu.sync_copy(dst_hbm.at[wid], dst_v)

  def chunk(j, carry):
    pltpu.sync_copy(ones_v, acc_o.at[src_v.at[j]], add=True)
    pltpu.sync_copy(ones_v, acc_i.at[dst_v.at[j]], add=True)
    return carry

  lax.fori_loop(0, NCHUNK_D, chunk, 0)
  plsc.subcore_barrier()

  base = s * ROWS_TILE
  pltpu.sync_copy(acc_o.at[pl.ds(base, ROWS_TILE)],
                  dego_hbm.at[c].at[pl.ds(base, ROWS_TILE)])
  pltpu.sync_copy(acc_i.at[pl.ds(base, ROWS_TILE)],
                  degi_hbm.at[c].at[pl.ds(base, ROWS_TILE)])


@functools.cache
def _degrees_sc():
  return pl.kernel(
      _deg_body,
      out_type=(
          jax.ShapeDtypeStruct((NC, N_PAD, 16), jnp.float32),
          jax.ShapeDtypeStruct((NC, N_PAD, 16), jnp.float32),
      ),
      mesh=_mesh(),
      scratch_types=[
          pltpu.VMEM_SHARED((N_PAD, 16), jnp.float32),
          pltpu.VMEM_SHARED((N_PAD, 16), jnp.float32),
          pltpu.VMEM((NCHUNK_D, KD), jnp.int32),
          pltpu.VMEM((NCHUNK_D, KD), jnp.int32),
          pltpu.VMEM((KD, 16), jnp.float32),
          pltpu.VMEM((ZROWS, 16), jnp.float32),
      ],
      compiler_params=pltpu.CompilerParams(use_tc_tiling_on_sc=False),
      name="gcn_degrees",
  )


def _agg_body(width, k, nchunk, nbuf, m_hbm, src_hbm, dst_hbm,
              out_hbm, acc, src_v, dst_v, rows, zbuf, *sems):
  c = lax.axis_index("c")
  s = lax.axis_index("s")
  wid = s * NC + c

  _zero_vmem_2d(zbuf, ZROWS, width)
  for r in range(ROWS_TILE // ZROWS):
    base = s * ROWS_TILE + r * ZROWS
    pltpu.sync_copy(zbuf, acc.at[pl.ds(base, ZROWS)])
  plsc.subcore_barrier()

  pltpu.sync_copy(src_hbm.at[wid], src_v)
  pltpu.sync_copy(dst_hbm.at[wid], dst_v)

  gsems = sems[0:2]
  ssems = sems[2:4]
  kh = k // 2
  dummy = m_hbm.at[pl.ds(0, kh)]

  def issue_g(h, b):
    pltpu.async_copy(m_hbm.at[src_v.at[h, 0]], rows.at[b, 0], gsems[b])
    pltpu.async_copy(m_hbm.at[src_v.at[h, 1]], rows.at[b, 1], gsems[b])

  def issue_s(h, b):
    pltpu.async_copy(rows.at[b, 0], acc.at[dst_v.at[h, 0]], ssems[b],
                     add=True)
    pltpu.async_copy(rows.at[b, 1], acc.at[dst_v.at[h, 1]], ssems[b],
                     add=True)

  def wait_g(b):
    pltpu.make_async_copy(dummy, rows.at[b, 0], gsems[b]).wait()
    pltpu.make_async_copy(dummy, rows.at[b, 1], gsems[b]).wait()

  def wait_s(b):
    pltpu.make_async_copy(dummy, rows.at[b, 0], ssems[b]).wait()
    pltpu.make_async_copy(dummy, rows.at[b, 1], ssems[b]).wait()

  issue_g(0, 0)
  issue_g(1, 1)
  wait_g(0)
  issue_s(0, 0)

  def slot_pair(i, carry):
    for h, b in ((2 * i + 1, 1), (2 * i + 2, 0)):
      nb = 1 - b

      @pl.when(h + 1 < nchunk)
      def _():
        wait_s(nb)
        issue_g(h + 1, nb)

      wait_g(b)
      issue_s(h, b)
    return carry

  lax.fori_loop(0, (nchunk - 1) // 2, slot_pair, 0)
  wait_s(1)
  wait_s(0)
  plsc.subcore_barrier()

  base = s * ROWS_TILE
  pltpu.sync_copy(acc.at[pl.ds(base, ROWS_TILE)],
                  out_hbm.at[c].at[pl.ds(base, ROWS_TILE)])


@functools.cache
def _make_agg(width, k, nchunk, nbuf, dtype):
  return pl.kernel(
      functools.partial(_agg_body, width, k, nchunk, nbuf),
      out_type=jax.ShapeDtypeStruct((NC, N_PAD, width), dtype),
      mesh=_mesh(),
      scratch_types=[
          pltpu.VMEM_SHARED((N_PAD, width), dtype),
          pltpu.VMEM((nchunk, 2, k // 2), jnp.int32),
          pltpu.VMEM((nchunk, 2, k // 2), jnp.int32),
          pltpu.VMEM((nbuf, 2, k // 2, width), dtype),
          pltpu.VMEM((ZROWS, width), dtype),
      ] + [pltpu.SemaphoreType.DMA] * (2 * nbuf),
      compiler_params=pltpu.CompilerParams(use_tc_tiling_on_sc=False),
      name=f"gcn_agg_{width}",
  )


_RB = 2000
_GRID = N_NODES // _RB


def _prep_tc(dego_ref, degi_ref, x_ref, s0_ref, nsrc_ref, ndst_ref):
  deg_o = dego_ref[0, :, 0:1] + dego_ref[1, :, 0:1]
  deg_i = degi_ref[0, :, 0:1] + degi_ref[1, :, 0:1]
  nsrc = 1.0 / jnp.sqrt(jnp.maximum(deg_o, 1.0))
  ndst = 1.0 / jnp.sqrt(jnp.maximum(deg_i, 1.0))
  nsrc_ref[...] = nsrc
  ndst_ref[...] = ndst
  s0_ref[...] = x_ref[...] * nsrc


def _mid_tc(p_ref, nsrc_ref, ndst_ref, w_ref, b_ref, s_ref):
  p = p_ref[0].astype(jnp.float32) + p_ref[1].astype(jnp.float32)
  h = p * ndst_ref[...]
  y = jnp.dot(h, w_ref[...], preferred_element_type=jnp.float32) + b_ref[...]
  s_ref[...] = jnp.maximum(y, 0.0) * nsrc_ref[...]


def _last_mid_tc(p_ref, nsrc_ref, ndst_ref, w_ref, b_ref, w5_ref, t_ref):
  p = p_ref[0].astype(jnp.float32) + p_ref[1].astype(jnp.float32)
  h = p * ndst_ref[...]
  y = jnp.dot(h, w_ref[...], preferred_element_type=jnp.float32) + b_ref[...]
  s = jnp.maximum(y, 0.0) * nsrc_ref[...]
  t_ref[...] = jnp.dot(s, w5_ref[...], preferred_element_type=jnp.float32)


def _final_tc(p_ref, ndst_ref, b5_ref, out_ref):
  o = (p_ref[0] + p_ref[1]) * ndst_ref[...]
  out_ref[...] = o[:, :N_CLASSES] + b5_ref[...]


def _row_spec(width):
  return pl.BlockSpec((_RB, width), lambda i: (i, 0))


def _part_spec(width):
  return pl.BlockSpec((NC, _RB, width), lambda i: (0, i, 0))


def _full_spec(shape):
  return pl.BlockSpec(shape, lambda i: tuple(0 for _ in shape))


def kernel(features, edge_index, W1, b1, W2, b2, W3, b3, W4, b4, W5, b5):
  src32 = edge_index[0].astype(jnp.int32)
  dst32 = edge_index[1].astype(jnp.int32)
  src = src32.reshape(NW, NCHUNK, 2, K // 2)
  dst = dst32.reshape(NW, NCHUNK, 2, K // 2)
  src_d = src32.reshape(NW, NCHUNK_D, KD)
  dst_d = dst32.reshape(NW, NCHUNK_D, KD)
  src_d2 = src32.reshape(NW, NCHUNK_D, 2, KD // 2)
  dst_d2 = dst32.reshape(NW, NCHUNK_D, 2, KD // 2)

  dego, degi = _degrees_sc()(src_d, dst_d)

  s0, nsrc, ndst = pl.pallas_call(
      _prep_tc,
      grid=(_GRID,),
      in_specs=[_part_spec(16), _part_spec(16), _row_spec(D_HID)],
      out_specs=[_row_spec(D_HID), _row_spec(1), _row_spec(1)],
      out_shape=[
          jax.ShapeDtypeStruct((N_NODES, D_HID), jnp.float32),
          jax.ShapeDtypeStruct((N_NODES, 1), jnp.float32),
          jax.ShapeDtypeStruct((N_NODES, 1), jnp.float32),
      ],
  )(dego, degi, features)

  mid = pl.pallas_call(
      _mid_tc,
      grid=(_GRID,),
      in_specs=[
          _part_spec(D_HID), _row_spec(1), _row_spec(1),
          _full_spec((D_HID, D_HID)), _full_spec((1, D_HID)),
      ],
      out_specs=_row_spec(D_HID),
      out_shape=jax.ShapeDtypeStruct((N_NODES, D_HID), jnp.float32),
  )

  last_mid = pl.pallas_call(
      _last_mid_tc,
      grid=(_GRID,),
      in_specs=[
          _part_spec(D_HID), _row_spec(1), _row_spec(1),
          _full_spec((D_HID, D_HID)), _full_spec((1, D_HID)),
          _full_spec((D_HID, D_OUT_PAD)),
      ],
      out_specs=_row_spec(D_OUT_PAD),
      out_shape=jax.ShapeDtypeStruct((N_NODES, D_OUT_PAD), jnp.float32),
  )

  W5p = jnp.zeros((D_HID, D_OUT_PAD), jnp.float32).at[:, :N_CLASSES].set(W5)

  agg128 = _make_agg(D_HID, K, NCHUNK, 2, jnp.float32)
  agg32 = _make_agg(D_OUT_PAD, KD, NCHUNK_D, 2, jnp.float32)
  p1 = agg128(s0, src, dst)
  s1 = mid(p1, nsrc, ndst, W1, b1.reshape(1, -1))
  p2 = agg128(s1, src, dst)
  s2 = mid(p2, nsrc, ndst, W2, b2.reshape(1, -1))
  p3 = agg128(s2, src, dst)
  s3 = mid(p3, nsrc, ndst, W3, b3.reshape(1, -1))
  p4 = agg128(s3, src, dst)
  t = last_mid(p4, nsrc, ndst, W4, b4.reshape(1, -1), W5p)
  p5 = agg32(t, src_d2, dst_d2)

  out = pl.pallas_call(
      _final_tc,
      grid=(_GRID,),
      in_specs=[
          _part_spec(D_OUT_PAD), _row_spec(1),
          _full_spec((1, N_CLASSES)),
      ],
      out_specs=_row_spec(N_CLASSES),
      out_shape=jax.ShapeDtypeStruct((N_NODES, N_CLASSES), jnp.float32),
  )(p5, ndst, b5.reshape(1, -1))
  return out


# speedup vs baseline: 1.1800x; 1.0131x over previous
import functools

import jax
import jax.numpy as jnp
from jax import lax
from jax.experimental import pallas as pl
from jax.experimental.pallas import tpu as pltpu
from jax.experimental.pallas import tpu_sc as plsc

N_NODES = 10000
N_PAD = 10240
N_EDGES = 320000
D_HID = 128
N_CLASSES = 21
D_OUT_PAD = 32

NC = 2
NS = 16
NW = NC * NS
E_TILE = N_EDGES // NW
K = 80
NCHUNK = E_TILE // K
KD = 400
NCHUNK_D = E_TILE // KD
ROWS_TILE = N_PAD // NS
ZROWS = 32

@functools.cache
def _mesh():
  return plsc.VectorSubcoreMesh(core_axis_name="c", subcore_axis_name="s",
                                num_cores=NC, num_subcores=NS)


def _zero_vmem_2d(ref, nrows, ncols):
  z16 = jnp.zeros((16,), jnp.float32)

  def body(j, carry):
    for q in range(ncols // 16):
      ref[j, pl.ds(q * 16, 16)] = z16
    return carry

  lax.fori_loop(0, nrows, body, 0)


def _deg_body(src_hbm, dst_hbm, dego_hbm, degi_hbm,
              acc_o, acc_i, src_v, dst_v, ones_v, zbuf):
  c = lax.axis_index("c")
  s = lax.axis_index("s")
  wid = s * NC + c

  _zero_vmem_2d(zbuf, ZROWS, 16)
  o16 = jnp.ones((16,), jnp.float32)

  def fill_ones(j, carry):
    ones_v[j, pl.ds(0, 16)] = o16
    return carry

  lax.fori_loop(0, KD, fill_ones, 0)

  for r in range(ROWS_TILE // ZROWS):
    base = s * ROWS_TILE + r * ZROWS
    pltpu.sync_copy(zbuf, acc_o.at[pl.ds(base, ZROWS)])
    pltpu.sync_copy(zbuf, acc_i.at[pl.ds(base, ZROWS)])
  plsc.subcore_barrier()

  pltpu.sync_copy(src_hbm.at[wid], src_v)
  pltpu.sync_copy(dst_hbm.at[wid], dst_v)

  def chunk(j, carry):
    pltpu.sync_copy(ones_v, acc_o.at[src_v.at[j]], add=True)
    pltpu.sync_copy(ones_v, acc_i.at[dst_v.at[j]], add=True)
    return carry

  lax.fori_loop(0, NCHUNK_D, chunk, 0)
  plsc.subcore_barrier()

  base = s * ROWS_TILE
  pltpu.sync_copy(acc_o.at[pl.ds(base, ROWS_TILE)],
                  dego_hbm.at[c].at[pl.ds(base, ROWS_TILE)])
  pltpu.sync_copy(acc_i.at[pl.ds(base, ROWS_TILE)],
                  degi_hbm.at[c].at[pl.ds(base, ROWS_TILE)])


@functools.cache
def _degrees_sc():
  return pl.kernel(
      _deg_body,
      out_type=(
          jax.ShapeDtypeStruct((NC, N_PAD, 16), jnp.float32),
          jax.ShapeDtypeStruct((NC, N_PAD, 16), jnp.float32),
      ),
      mesh=_mesh(),
      scratch_types=[
          pltpu.VMEM_SHARED((N_PAD, 16), jnp.float32),
          pltpu.VMEM_SHARED((N_PAD, 16), jnp.float32),
          pltpu.VMEM((NCHUNK_D, KD), jnp.int32),
          pltpu.VMEM((NCHUNK_D, KD), jnp.int32),
          pltpu.VMEM((KD, 16), jnp.float32),
          pltpu.VMEM((ZROWS, 16), jnp.float32),
      ],
      compiler_params=pltpu.CompilerParams(use_tc_tiling_on_sc=False),
      name="gcn_degrees",
  )


def _agg_body(width, k, nchunk, nbuf, m_hbm, src_hbm, dst_hbm, z_hbm,
              out_hbm, acc, src_v, dst_v, rows, zbuf, *sems):
  c = lax.axis_index("c")
  s = lax.axis_index("s")
  wid = s * NC + c

  pltpu.sync_copy(z_hbm, zbuf)
  for r in range(ROWS_TILE // ZROWS):
    base = s * ROWS_TILE + r * ZROWS
    pltpu.sync_copy(zbuf, acc.at[pl.ds(base, ZROWS)])
  plsc.subcore_barrier()

  pltpu.sync_copy(src_hbm.at[wid], src_v)
  pltpu.sync_copy(dst_hbm.at[wid], dst_v)

  gsems = sems[0:2]
  ssems = sems[2:4]
  kh = k // 2
  dummy = m_hbm.at[pl.ds(0, kh)]

  def issue_g(h, b):
    pltpu.async_copy(m_hbm.at[src_v.at[h, 0]], rows.at[b, 0], gsems[b])
    pltpu.async_copy(m_hbm.at[src_v.at[h, 1]], rows.at[b, 1], gsems[b])

  def issue_s(h, b):
    pltpu.async_copy(rows.at[b, 0], acc.at[dst_v.at[h, 0]], ssems[b],
                     add=True)
    pltpu.async_copy(rows.at[b, 1], acc.at[dst_v.at[h, 1]], ssems[b],
                     add=True)

  def wait_g(b):
    pltpu.make_async_copy(dummy, rows.at[b, 0], gsems[b]).wait()
    pltpu.make_async_copy(dummy, rows.at[b, 1], gsems[b]).wait()

  def wait_s(b):
    pltpu.make_async_copy(dummy, rows.at[b, 0], ssems[b]).wait()
    pltpu.make_async_copy(dummy, rows.at[b, 1], ssems[b]).wait()

  issue_g(0, 0)
  issue_g(1, 1)
  wait_g(0)
  issue_s(0, 0)

  def slot_pair(i, carry):
    for h, b in ((2 * i + 1, 1), (2 * i + 2, 0)):
      nb = 1 - b

      @pl.when(h + 1 < nchunk)
      def _():
        wait_s(nb)
        issue_g(h + 1, nb)

      wait_g(b)
      issue_s(h, b)
    return carry

  lax.fori_loop(0, (nchunk - 1) // 2, slot_pair, 0)
  wait_s(1)
  wait_s(0)
  plsc.subcore_barrier()

  base = s * ROWS_TILE
  pltpu.sync_copy(acc.at[pl.ds(base, ROWS_TILE)],
                  out_hbm.at[c].at[pl.ds(base, ROWS_TILE)])


@functools.cache
def _make_agg(width, k, nchunk, nbuf, dtype):
  return pl.kernel(
      functools.partial(_agg_body, width, k, nchunk, nbuf),
      out_type=jax.ShapeDtypeStruct((NC, N_PAD, width), dtype),
      mesh=_mesh(),
      scratch_types=[
          pltpu.VMEM_SHARED((N_PAD, width), dtype),
          pltpu.VMEM((nchunk, 2, k // 2), jnp.int32),
          pltpu.VMEM((nchunk, 2, k // 2), jnp.int32),
          pltpu.VMEM((nbuf, 2, k // 2, width), dtype),
          pltpu.VMEM((ZROWS, width), dtype),
      ] + [pltpu.SemaphoreType.DMA] * (2 * nbuf),
      compiler_params=pltpu.CompilerParams(use_tc_tiling_on_sc=False),
      name=f"gcn_agg_{width}",
  )


_RB = 2000
_GRID = N_NODES // _RB


def _prep_tc(dego_ref, degi_ref, x_ref, s0_ref, nsrc_ref, ndst_ref):
  deg_o = dego_ref[0, :, 0:1] + dego_ref[1, :, 0:1]
  deg_i = degi_ref[0, :, 0:1] + degi_ref[1, :, 0:1]
  nsrc = 1.0 / jnp.sqrt(jnp.maximum(deg_o, 1.0))
  ndst = 1.0 / jnp.sqrt(jnp.maximum(deg_i, 1.0))
  nsrc_ref[...] = nsrc
  ndst_ref[...] = ndst
  s0_ref[...] = (x_ref[...] * nsrc).astype(jnp.bfloat16)


def _mid_tc(p_ref, nsrc_ref, ndst_ref, w_ref, b_ref, s_ref):
  p = p_ref[0].astype(jnp.float32) + p_ref[1].astype(jnp.float32)
  h = p * ndst_ref[...]
  y = jnp.dot(h, w_ref[...], preferred_element_type=jnp.float32) + b_ref[...]
  s_ref[...] = (jnp.maximum(y, 0.0) * nsrc_ref[...]).astype(jnp.bfloat16)


def _last_mid_tc(p_ref, nsrc_ref, ndst_ref, w_ref, b_ref, w5_ref, t_ref):
  p = p_ref[0].astype(jnp.float32) + p_ref[1].astype(jnp.float32)
  h = p * ndst_ref[...]
  y = jnp.dot(h, w_ref[...], preferred_element_type=jnp.float32) + b_ref[...]
  s = jnp.maximum(y, 0.0) * nsrc_ref[...]
  t_ref[...] = jnp.dot(s, w5_ref[...], preferred_element_type=jnp.float32)


def _final_tc(p_ref, ndst_ref, b5_ref, out_ref):
  o = (p_ref[0] + p_ref[1]) * ndst_ref[...]
  out_ref[...] = o[:, :N_CLASSES] + b5_ref[...]


def _row_spec(width):
  return pl.BlockSpec((_RB, width), lambda i: (i, 0))


def _part_spec(width):
  return pl.BlockSpec((NC, _RB, width), lambda i: (0, i, 0))


def _full_spec(shape):
  return pl.BlockSpec(shape, lambda i: tuple(0 for _ in shape))


def kernel(features, edge_index, W1, b1, W2, b2, W3, b3, W4, b4, W5, b5):
  src32 = edge_index[0].astype(jnp.int32)
  dst32 = edge_index[1].astype(jnp.int32)
  src = src32.reshape(NW, NCHUNK, 2, K // 2)
  dst = dst32.reshape(NW, NCHUNK, 2, K // 2)
  src_d = src32.reshape(NW, NCHUNK_D, KD)
  dst_d = dst32.reshape(NW, NCHUNK_D, KD)
  src_d2 = src32.reshape(NW, NCHUNK_D, 2, KD // 2)
  dst_d2 = dst32.reshape(NW, NCHUNK_D, 2, KD // 2)

  dego, degi = _degrees_sc()(src_d, dst_d)

  s0, nsrc, ndst = pl.pallas_call(
      _prep_tc,
      grid=(_GRID,),
      in_specs=[_part_spec(16), _part_spec(16), _row_spec(D_HID)],
      out_specs=[_row_spec(D_HID), _row_spec(1), _row_spec(1)],
      out_shape=[
          jax.ShapeDtypeStruct((N_NODES, D_HID), jnp.bfloat16),
          jax.ShapeDtypeStruct((N_NODES, 1), jnp.float32),
          jax.ShapeDtypeStruct((N_NODES, 1), jnp.float32),
      ],
  )(dego, degi, features)

  mid = pl.pallas_call(
      _mid_tc,
      grid=(_GRID,),
      in_specs=[
          _part_spec(D_HID), _row_spec(1), _row_spec(1),
          _full_spec((D_HID, D_HID)), _full_spec((1, D_HID)),
      ],
      out_specs=_row_spec(D_HID),
      out_shape=jax.ShapeDtypeStruct((N_NODES, D_HID), jnp.bfloat16),
  )

  last_mid = pl.pallas_call(
      _last_mid_tc,
      grid=(_GRID,),
      in_specs=[
          _part_spec(D_HID), _row_spec(1), _row_spec(1),
          _full_spec((D_HID, D_HID)), _full_spec((1, D_HID)),
          _full_spec((D_HID, D_OUT_PAD)),
      ],
      out_specs=_row_spec(D_OUT_PAD),
      out_shape=jax.ShapeDtypeStruct((N_NODES, D_OUT_PAD), jnp.float32),
  )

  W5p = jnp.zeros((D_HID, D_OUT_PAD), jnp.float32).at[:, :N_CLASSES].set(W5)

  z128 = jnp.zeros((ZROWS, D_HID), jnp.bfloat16)
  z32 = jnp.zeros((ZROWS, D_OUT_PAD), jnp.float32)
  agg128 = _make_agg(D_HID, K, NCHUNK, 2, jnp.bfloat16)
  agg32 = _make_agg(D_OUT_PAD, KD, NCHUNK_D, 2, jnp.float32)
  p1 = agg128(s0, src, dst, z128)
  s1 = mid(p1, nsrc, ndst, W1, b1.reshape(1, -1))
  p2 = agg128(s1, src, dst, z128)
  s2 = mid(p2, nsrc, ndst, W2, b2.reshape(1, -1))
  p3 = agg128(s2, src, dst, z128)
  s3 = mid(p3, nsrc, ndst, W3, b3.reshape(1, -1))
  p4 = agg128(s3, src, dst, z128)
  t = last_mid(p4, nsrc, ndst, W4, b4.reshape(1, -1), W5p)
  p5 = agg32(t, src_d2, dst_d2, z32)

  out = pl.pallas_call(
      _final_tc,
      grid=(_GRID,),
      in_specs=[
          _part_spec(D_OUT_PAD), _row_spec(1),
          _full_spec((1, N_CLASSES)),
      ],
      out_specs=_row_spec(N_CLASSES),
      out_shape=jax.ShapeDtypeStruct((N_NODES, N_CLASSES), jnp.float32),
  )(p5, ndst, b5.reshape(1, -1))
  return out
